# Initial kernel scaffold; baseline (speedup 1.0000x reference)
import jax
import jax.numpy as jnp
from jax.experimental import pallas as pl


def kernel(x, table):
    raise NotImplementedError("write your pallas kernel here")


# speedup vs baseline: 1.8936x; 1.8936x over previous
import functools

import jax
import jax.numpy as jnp
from jax import lax
from jax.experimental import pallas as pl
from jax.experimental.pallas import tpu as pltpu
from jax.experimental.pallas import tpu_sc as plsc

VOCAB = 1000
EMBED_DIM = 32
BATCH = 16384
FIELDS = 100

_B = BATCH * FIELDS
_INFO = plsc.get_sparse_core_info()
_NC = _INFO.num_cores
_NS = _INFO.num_subcores
_NW = _NC * _NS
_BPW = _B // _NW

_K = 8
_C = _K * 128
_CHUNKS = _BPW // _C
_XROWS_PER_W = _BPW // 128


def _emb_body(table_hbm, x_hbm, out_hbm, idx_v, rows_v, sem):
    wid = lax.axis_index("s") * _NC + lax.axis_index("c")
    xrow0 = wid * _XROWS_PER_W
    base = wid * _BPW

    def chunk(g, carry):
        pltpu.sync_copy(x_hbm.at[pl.ds(xrow0 + g * _K, _K)], idx_v)
        cps = [
            pltpu.async_copy(
                table_hbm.at[idx_v.at[k]],
                rows_v.at[pl.ds(k * 128, 128)],
                sem,
            )
            for k in range(_K)
        ]
        for cp in cps:
            cp.wait()
        pltpu.sync_copy(rows_v, out_hbm.at[pl.ds(base + g * _C, _C)])
        return carry

    lax.fori_loop(0, _CHUNKS, chunk, 0)


_emb = pl.kernel(
    _emb_body,
    out_type=jax.ShapeDtypeStruct((_B, EMBED_DIM), jnp.float32),
    mesh=plsc.VectorSubcoreMesh(core_axis_name="c", subcore_axis_name="s"),
    compiler_params=pltpu.CompilerParams(use_tc_tiling_on_sc=False),
    scratch_types=[
        pltpu.VMEM((_K, 128), jnp.int32),
        pltpu.VMEM((_C, EMBED_DIM), jnp.float32),
        pltpu.SemaphoreType.DMA,
    ],
)


def kernel(x, table):
    xf = x.reshape(_B // 128, 128).astype(jnp.int32)
    out = _emb(table, xf)
    return out.reshape(BATCH, FIELDS, EMBED_DIM)


# speedup vs baseline: 18.4932x; 9.7661x over previous
import functools

import jax
import jax.numpy as jnp
from jax import lax
from jax.experimental import pallas as pl
from jax.experimental.pallas import tpu as pltpu
from jax.experimental.pallas import tpu_sc as plsc

VOCAB = 1000
EMBED_DIM = 32
BATCH = 16384
FIELDS = 100

_INFO = plsc.get_sparse_core_info()
_NC = _INFO.num_cores
_NS = _INFO.num_subcores
_NW = _NC * _NS
_COLS = BATCH // _NW
_NVEC = _COLS // 16


def _emb_body(tt_hbm, xt_hbm, out_hbm, tab_v, idx_v, out_v, sem):
    wid = lax.axis_index("s") * _NC + lax.axis_index("c")
    col0 = wid * _COLS

    pltpu.sync_copy(tt_hbm, tab_v)

    dvecs = [jnp.full((16,), d, jnp.int32) for d in range(EMBED_DIM)]

    def field(f, carry):
        pltpu.sync_copy(xt_hbm.at[f, pl.ds(col0, _COLS)], idx_v)

        def group(i, c2):
            idx16 = idx_v[pl.ds(i * 16, 16)]
            for d in range(EMBED_DIM):
                vals = plsc.load_gather(tab_v, [dvecs[d], idx16])
                out_v[d, pl.ds(i * 16, 16)] = vals
            return c2

        lax.fori_loop(0, _NVEC, group, 0)
        pltpu.sync_copy(out_v, out_hbm.at[f, :, pl.ds(col0, _COLS)])
        return carry

    lax.fori_loop(0, FIELDS, field, 0)


_emb = pl.kernel(
    _emb_body,
    out_type=jax.ShapeDtypeStruct((FIELDS, EMBED_DIM, BATCH), jnp.float32),
    mesh=plsc.VectorSubcoreMesh(core_axis_name="c", subcore_axis_name="s"),
    compiler_params=pltpu.CompilerParams(needs_layout_passes=False),
    scratch_types=[
        pltpu.VMEM((EMBED_DIM, VOCAB), jnp.float32),
        pltpu.VMEM((_COLS,), jnp.int32),
        pltpu.VMEM((EMBED_DIM, _COLS), jnp.float32),
        pltpu.SemaphoreType.DMA,
    ],
)


def kernel(x, table):
    xt = x.T.astype(jnp.int32)
    tt = table.T
    outt = _emb(tt, xt)
    return outt.transpose(2, 0, 1)


# speedup vs baseline: 57.4926x; 3.1089x over previous
import functools

import jax
import jax.numpy as jnp
from jax import lax
from jax.experimental import pallas as pl
from jax.experimental.pallas import tpu as pltpu
from jax.experimental.pallas import tpu_sc as plsc

VOCAB = 1000
EMBED_DIM = 32
BATCH = 16384
FIELDS = 100

_INFO = plsc.get_sparse_core_info()
_NC = _INFO.num_cores
_NS = _INFO.num_subcores
_NW = _NC * _NS
_COLS = BATCH // _NW
_NVEC = _COLS // 16


def _emb_body(tt_hbm, xt_hbm, out_hbm,
              tab_v, idx_a, idx_b, out_a, out_b,
              sem_ia, sem_ib, sem_oa, sem_ob):
    wid = lax.axis_index("s") * _NC + lax.axis_index("c")
    col0 = wid * _COLS

    pltpu.sync_copy(tt_hbm, tab_v)

    dvecs = [jnp.full((16,), d, jnp.int32) for d in range(EMBED_DIM)]

    def compute(idx_v, out_v):
        @plsc.parallel_loop(0, _NVEC, 1, unroll=2)
        def group(i):
            idx16 = idx_v[pl.ds(i * 16, 16)]
            vals = [
                plsc.load_gather(tab_v, [dvecs[d], idx16])
                for d in range(EMBED_DIM)
            ]
            for d in range(EMBED_DIM):
                out_v[d, pl.ds(i * 16, 16)] = vals[d]

    def idx_start(f, buf, sem):
        pltpu.async_copy(xt_hbm.at[f, pl.ds(col0, _COLS)], buf, sem)

    def idx_wait(f, buf, sem):
        pltpu.make_async_copy(xt_hbm.at[f, pl.ds(col0, _COLS)], buf, sem).wait()

    def out_start(f, buf, sem):
        pltpu.async_copy(buf, out_hbm.at[f, :, pl.ds(col0, _COLS)], sem)

    def out_wait(f, buf, sem):
        pltpu.make_async_copy(buf, out_hbm.at[f, :, pl.ds(col0, _COLS)], sem).wait()

    idx_start(0, idx_a, sem_ia)

    def pair(g, carry):
        fa = 2 * g
        fb = fa + 1
        idx_start(fb, idx_b, sem_ib)
        idx_wait(fa, idx_a, sem_ia)

        @pl.when(g > 0)
        def _():
            out_wait(fa, out_a, sem_oa)

        compute(idx_a, out_a)
        out_start(fa, out_a, sem_oa)

        @pl.when(fb + 1 < FIELDS)
        def _():
            idx_start(fb + 1, idx_a, sem_ia)

        idx_wait(fb, idx_b, sem_ib)

        @pl.when(g > 0)
        def _():
            out_wait(fb, out_b, sem_ob)

        compute(idx_b, out_b)
        out_start(fb, out_b, sem_ob)
        return carry

    lax.fori_loop(0, FIELDS // 2, pair, 0)
    out_wait(FIELDS - 2, out_a, sem_oa)
    out_wait(FIELDS - 1, out_b, sem_ob)


_emb = pl.kernel(
    _emb_body,
    out_type=jax.ShapeDtypeStruct((FIELDS, EMBED_DIM, BATCH), jnp.float32),
    mesh=plsc.VectorSubcoreMesh(core_axis_name="c", subcore_axis_name="s"),
    compiler_params=pltpu.CompilerParams(needs_layout_passes=False),
    scratch_types=[
        pltpu.VMEM((EMBED_DIM, VOCAB), jnp.float32),
        pltpu.VMEM((_COLS,), jnp.int32),
        pltpu.VMEM((_COLS,), jnp.int32),
        pltpu.VMEM((EMBED_DIM, _COLS), jnp.float32),
        pltpu.VMEM((EMBED_DIM, _COLS), jnp.float32),
        pltpu.SemaphoreType.DMA,
        pltpu.SemaphoreType.DMA,
        pltpu.SemaphoreType.DMA,
        pltpu.SemaphoreType.DMA,
    ],
)


def kernel(x, table):
    xt = x.T.astype(jnp.int32)
    tt = table.T
    outt = _emb(tt, xt)
    return outt.transpose(2, 0, 1)
